# baseline (speedup 1.0000x reference)
import functools

import jax
import jax.numpy as jnp
from jax import lax
from jax.experimental import pallas as pl
from jax.experimental.pallas import tpu as pltpu
from jax.experimental.pallas import tpu_sc as plsc

f32 = jnp.float32
i32 = jnp.int32

_NW = 32
_CHUNK = 128


def _sigmoid(x):
    return 1.0 / (1.0 + jnp.exp(-x))


def _silu(x):
    return x * _sigmoid(x)


def _softplus(x):
    return jnp.maximum(x, 0.0) + jnp.log(1.0 + jnp.exp(-jnp.abs(x)))


def _elu(x):
    return jnp.where(x > 0, x, jnp.exp(jnp.minimum(x, 0.0)) - 1.0)


def _softmax(x):
    e = jnp.exp(x - jnp.max(x, axis=1, keepdims=True))
    return e / jnp.sum(e, axis=1, keepdims=True)


def _bn_cols(x):
    mu = jnp.mean(x, axis=0, keepdims=True)
    var = jnp.mean((x - mu) * (x - mu), axis=0, keepdims=True)
    return (x - mu) / jnp.sqrt(var + 1e-5)


def _mm_stats(x, w, b, bn):
    n = x.shape[0]
    dout = w.shape[1]
    grid = n // bn

    def kern(x_ref, w_ref, b_ref, ss_ref, sq_ref):
        @pl.when(pl.program_id(0) == 0)
        def _():
            ss_ref[...] = jnp.zeros_like(ss_ref)
            sq_ref[...] = jnp.zeros_like(sq_ref)

        t = jnp.dot(x_ref[...], w_ref[...],
                    preferred_element_type=f32) + b_ref[...]
        ss_ref[...] += jnp.sum(t, axis=0, keepdims=True)
        sq_ref[...] += jnp.sum(t * t, axis=0, keepdims=True)

    return pl.pallas_call(
        kern, grid=(grid,),
        in_specs=[pl.BlockSpec((bn, x.shape[1]), lambda i: (i, 0)),
                  pl.BlockSpec(w.shape, lambda i: (0, 0)),
                  pl.BlockSpec(b.shape, lambda i: (0, 0))],
        out_specs=[pl.BlockSpec((1, dout), lambda i: (0, 0)),
                   pl.BlockSpec((1, dout), lambda i: (0, 0))],
        out_shape=[jax.ShapeDtypeStruct((1, dout), f32),
                   jax.ShapeDtypeStruct((1, dout), f32)])(x, w, b)


def _bn_affine(ssum, ssq, n, g, bt):
    mu = ssum / n
    var = ssq / n - mu * mu
    a = g / jnp.sqrt(var + 1e-5)
    return a, bt - a * mu


def _emb_apply(x, w, b, a, c, npad, bn):
    n = x.shape[0]
    d = w.shape[1]
    grid = n // bn

    def kern(x_ref, w_ref, b_ref, a_ref, c_ref, o_ref):
        t = jnp.dot(x_ref[...], w_ref[...],
                    preferred_element_type=f32) + b_ref[...]
        o_ref[...] = jnp.pad(_silu(a_ref[...] * t + c_ref[...]),
                             ((0, 0), (0, npad - d)))

    return pl.pallas_call(
        kern, grid=(grid,),
        in_specs=[pl.BlockSpec((bn, x.shape[1]), lambda i: (i, 0)),
                  pl.BlockSpec(w.shape, lambda i: (0, 0)),
                  pl.BlockSpec(b.shape, lambda i: (0, 0)),
                  pl.BlockSpec(a.shape, lambda i: (0, 0)),
                  pl.BlockSpec(c.shape, lambda i: (0, 0))],
        out_specs=pl.BlockSpec((bn, npad), lambda i: (i, 0)),
        out_shape=jax.ShapeDtypeStruct((n, npad), f32))(x, w, b, a, c)


def _y_block(ea, ns, nd, wsrc, wdst, we, ae, ce, wc, bc):
    t = jnp.dot(ea, we, preferred_element_type=f32)
    hm = _silu(ae * t + ce)
    return (jnp.dot(ns, wsrc, preferred_element_type=f32) +
            jnp.dot(nd, wdst, preferred_element_type=f32) +
            jnp.dot(hm, wc, preferred_element_type=f32) + bc)


def _edge_stats(ea, gs, gd, wsrc, wdst, we, ae, ce, wc, bc, bn):
    e, de = ea.shape
    npad = gs.shape[1]
    p2 = wc.shape[1]
    grid = e // bn

    def kern(ea_ref, gs_ref, gd_ref, w1_ref, w2_ref, we_ref, ae_ref, ce_ref,
             wc_ref, bc_ref, ss_ref, sq_ref):
        @pl.when(pl.program_id(0) == 0)
        def _():
            ss_ref[...] = jnp.zeros_like(ss_ref)
            sq_ref[...] = jnp.zeros_like(sq_ref)

        y = _y_block(ea_ref[...], gs_ref[...], gd_ref[...], w1_ref[...],
                     w2_ref[...], we_ref[...], ae_ref[...], ce_ref[...],
                     wc_ref[...], bc_ref[...])
        ss_ref[...] += jnp.sum(y, axis=0, keepdims=True)
        sq_ref[...] += jnp.sum(y * y, axis=0, keepdims=True)

    return pl.pallas_call(
        kern, grid=(grid,),
        in_specs=[pl.BlockSpec((bn, de), lambda i: (i, 0)),
                  pl.BlockSpec((bn, npad), lambda i: (i, 0)),
                  pl.BlockSpec((bn, npad), lambda i: (i, 0)),
                  pl.BlockSpec(wsrc.shape, lambda i: (0, 0)),
                  pl.BlockSpec(wdst.shape, lambda i: (0, 0)),
                  pl.BlockSpec(we.shape, lambda i: (0, 0)),
                  pl.BlockSpec(ae.shape, lambda i: (0, 0)),
                  pl.BlockSpec(ce.shape, lambda i: (0, 0)),
                  pl.BlockSpec(wc.shape, lambda i: (0, 0)),
                  pl.BlockSpec(bc.shape, lambda i: (0, 0))],
        out_specs=[pl.BlockSpec((1, p2), lambda i: (0, 0)),
                   pl.BlockSpec((1, p2), lambda i: (0, 0))],
        out_shape=[jax.ShapeDtypeStruct((1, p2), f32),
                   jax.ShapeDtypeStruct((1, p2), f32)])(
            ea, gs, gd, wsrc, wdst, we, ae, ce, wc, bc)


def _edge_update(ea, gs, gd, wsrc, wdst, we, ae, ce, wc, bc, am, cm, a2, c2,
                 dpu, bn):
    e, de = ea.shape
    npad = gs.shape[1]
    grid = e // bn

    nh = (dpu + 127) // 128

    def kern(ea_ref, gs_ref, gd_ref, w1_ref, w2_ref, we_ref, ae_ref, ce_ref,
             wc_ref, bc_ref, am_ref, cm_ref, a2_ref, c2_ref, u_ref):
        y = _y_block(ea_ref[...], gs_ref[...], gd_ref[...], w1_ref[...],
                     w2_ref[...], we_ref[...], ae_ref[...], ce_ref[...],
                     wc_ref[...], bc_ref[...])
        y1 = y[:, :dpu]
        y2 = y[:, dpu:]
        u = (_silu(am_ref[...] * y1 + cm_ref[...]) *
             _softplus(a2_ref[...] * y2 + c2_ref[...]))
        if nh * 128 > dpu:
            u = jnp.pad(u, ((0, 0), (0, nh * 128 - dpu)))
        for h in range(nh):
            u_ref[h, :, :] = u[:, h * 128:(h + 1) * 128]

    return pl.pallas_call(
        kern, grid=(grid,),
        in_specs=[pl.BlockSpec((bn, de), lambda i: (i, 0)),
                  pl.BlockSpec((bn, npad), lambda i: (i, 0)),
                  pl.BlockSpec((bn, npad), lambda i: (i, 0)),
                  pl.BlockSpec(wsrc.shape, lambda i: (0, 0)),
                  pl.BlockSpec(wdst.shape, lambda i: (0, 0)),
                  pl.BlockSpec(we.shape, lambda i: (0, 0)),
                  pl.BlockSpec(ae.shape, lambda i: (0, 0)),
                  pl.BlockSpec(ce.shape, lambda i: (0, 0)),
                  pl.BlockSpec(wc.shape, lambda i: (0, 0)),
                  pl.BlockSpec(bc.shape, lambda i: (0, 0)),
                  pl.BlockSpec(am.shape, lambda i: (0, 0)),
                  pl.BlockSpec(cm.shape, lambda i: (0, 0)),
                  pl.BlockSpec(a2.shape, lambda i: (0, 0)),
                  pl.BlockSpec(c2.shape, lambda i: (0, 0))],
        out_specs=pl.BlockSpec((nh, bn, 128), lambda i: (0, i, 0)),
        out_shape=jax.ShapeDtypeStruct((nh, e, 128), f32))(
            ea, gs, gd, wsrc, wdst, we, ae, ce, wc, bc, am, cm, a2, c2)


def _agg_stats(partials, bn):
    nh, _, n, _ = partials.shape
    dpu = nh * 128
    grid = n // bn

    def kern(p_ref, ss_ref, sq_ref):
        @pl.when(pl.program_id(0) == 0)
        def _():
            ss_ref[...] = jnp.zeros_like(ss_ref)
            sq_ref[...] = jnp.zeros_like(sq_ref)

        agg = jnp.concatenate(
            [p_ref[h, 0] + p_ref[h, 1] for h in range(nh)], axis=1)
        ss_ref[...] += jnp.sum(agg, axis=0, keepdims=True)
        sq_ref[...] += jnp.sum(agg * agg, axis=0, keepdims=True)

    return pl.pallas_call(
        kern, grid=(grid,),
        in_specs=[pl.BlockSpec((nh, 2, bn, 128), lambda i: (0, 0, i, 0))],
        out_specs=[pl.BlockSpec((1, dpu), lambda i: (0, 0)),
                   pl.BlockSpec((1, dpu), lambda i: (0, 0))],
        out_shape=[jax.ShapeDtypeStruct((1, dpu), f32),
                   jax.ShapeDtypeStruct((1, dpu), f32)])(partials)


def _node_update(partials, node, aa, ca, d, bn):
    nh, _, n, _ = partials.shape
    npad = node.shape[1]
    grid = n // bn

    def kern(p_ref, nd_ref, aa_ref, ca_ref, o_ref):
        agg = jnp.concatenate(
            [p_ref[h, 0] + p_ref[h, 1] for h in range(nh)], axis=1)[:, :d]
        res = _softplus(aa_ref[...] * agg + ca_ref[...] + nd_ref[:, :d])
        o_ref[...] = jnp.pad(res, ((0, 0), (0, npad - d)))

    return pl.pallas_call(
        kern, grid=(grid,),
        in_specs=[pl.BlockSpec((nh, 2, bn, 128), lambda i: (0, 0, i, 0)),
                  pl.BlockSpec((bn, npad), lambda i: (i, 0)),
                  pl.BlockSpec(aa.shape, lambda i: (0, 0)),
                  pl.BlockSpec(ca.shape, lambda i: (0, 0))],
        out_specs=pl.BlockSpec((bn, npad), lambda i: (i, 0)),
        out_shape=jax.ShapeDtypeStruct((n, npad), f32))(partials, node, aa, ca)


def _pool_sums(node, gid2d, n_graphs, bn):
    n, d = node.shape
    grid = n // bn

    def kern(g_ref, v_ref, s_ref, c_ref):
        @pl.when(pl.program_id(0) == 0)
        def _():
            s_ref[...] = jnp.zeros_like(s_ref)
            c_ref[...] = jnp.zeros_like(c_ref)

        gid = g_ref[...]
        oh = (gid == lax.broadcasted_iota(i32, (bn, n_graphs), 1)).astype(f32)
        s_ref[...] += lax.dot_general(oh, v_ref[...], (((0,), (0,)), ((), ())),
                                      preferred_element_type=f32,
                                      precision=lax.Precision.HIGHEST)
        c_ref[...] += lax.dot_general(
            oh, jnp.ones((bn, 1), f32), (((0,), (0,)), ((), ())),
            preferred_element_type=f32, precision=lax.Precision.HIGHEST)

    return pl.pallas_call(
        kern, grid=(grid,),
        in_specs=[pl.BlockSpec((bn, 1), lambda i: (i, 0)),
                  pl.BlockSpec((bn, d), lambda i: (i, 0))],
        out_specs=[pl.BlockSpec((n_graphs, d), lambda i: (0, 0)),
                   pl.BlockSpec((n_graphs, 1), lambda i: (0, 0))],
        out_shape=[jax.ShapeDtypeStruct((n_graphs, d), f32),
                   jax.ShapeDtypeStruct((n_graphs, 1), f32)])(gid2d, node)


def _head(va_sum, va_cnt, vs_sum, vs_cnt, da, ds, plist):
    g = va_sum.shape[0]

    def kern(vas_ref, vac_ref, vss_ref, vsc_ref,
             atw_ref, atb_ref, atg_ref, att_ref, abg_ref, abb_ref,
             f0w_ref, f0b_ref, f0g_ref, f0t_ref,
             f1w_ref, f1b_ref, f1g_ref, f1t_ref,
             f2w_ref, f2b_ref, f2g_ref, f2t_ref,
             paw_ref, pab_ref, pag_ref, pat_ref,
             psw_ref, psb_ref, psg_ref, pst_ref,
             ptw_ref, ptb_ref, o_ref):
        va_s = vas_ref[:, :da] / jnp.maximum(vac_ref[...], 1.0)
        vs_s = vss_ref[:, :ds] / jnp.maximum(vsc_ref[...], 1.0)
        vt = jnp.concatenate([va_s, vs_s], axis=1)
        h = jnp.dot(vt, atw_ref[...], preferred_element_type=f32) + atb_ref[...]
        h = _elu(atg_ref[...] * _bn_cols(h) + att_ref[...]) * vt
        vt2 = abg_ref[...] * _bn_cols(h) + abb_ref[...]
        x = vt2
        for ww, bb, gg, tt in ((f0w_ref, f0b_ref, f0g_ref, f0t_ref),
                               (f1w_ref, f1b_ref, f1g_ref, f1t_ref),
                               (f2w_ref, f2b_ref, f2g_ref, f2t_ref)):
            x = jnp.dot(x, ww[...], preferred_element_type=f32) + bb[...]
            x = _silu(gg[...] * _bn_cols(x) + tt[...])
        ya = jnp.dot(x, paw_ref[...], preferred_element_type=f32) + pab_ref[...]
        ca = _softmax(pag_ref[...] * _bn_cols(ya) + pat_ref[...])
        ys = jnp.dot(x, psw_ref[...], preferred_element_type=f32) + psb_ref[...]
        cs = _softmax(psg_ref[...] * _bn_cols(ys) + pst_ref[...])
        t = jnp.dot(x, ptw_ref[...], preferred_element_type=f32) + ptb_ref[...]
        o_ref[...] = jnp.concatenate([ca, cs, t], axis=1)

    args = [va_sum, va_cnt, vs_sum, vs_cnt] + plist
    return pl.pallas_call(
        kern,
        in_specs=[pl.BlockSpec(a.shape, None) for a in args],
        out_specs=pl.BlockSpec((g, 10), None),
        out_shape=jax.ShapeDtypeStruct((g, 10), f32))(*args)


@functools.lru_cache(None)
def _make_gather(e, n, p):
    per = e // _NW
    full = per // _CHUNK
    rem = per - full * _CHUNK
    assert full % 2 == 1 and full >= 3
    nloop = (full - 1) // 2
    mesh = plsc.VectorSubcoreMesh(core_axis_name="c", subcore_axis_name="s")

    scratch = [pltpu.VMEM((_CHUNK,), i32), pltpu.VMEM((_CHUNK, p), f32),
               pltpu.VMEM((_CHUNK,), i32), pltpu.VMEM((_CHUNK, p), f32)]
    if rem:
        scratch += [pltpu.VMEM((rem,), i32), pltpu.VMEM((rem, p), f32)]
    scratch += [pltpu.SemaphoreType.DMA, pltpu.SemaphoreType.DMA,
                pltpu.SemaphoreType.DMA]

    @functools.partial(
        pl.kernel, mesh=mesh,
        out_type=[jax.ShapeDtypeStruct((e, p), f32),
                  jax.ShapeDtypeStruct((e, p), f32)],
        scratch_types=scratch,
        compiler_params=pltpu.CompilerParams(use_tc_tiling_on_sc=False))
    def kern(tab_hbm, src_hbm, dst_hbm, gs_hbm, gd_hbm, *scr):
        if rem:
            (i0, r0, i1, r1, ir, rr, sem0, sem1, semr) = scr
        else:
            (i0, r0, i1, r1, sem0, sem1, semr) = scr
        wid = lax.axis_index("s") * 2 + lax.axis_index("c")
        base0 = wid * per

        def table_pass(idx_hbm, out_hbm):
            def start(j, ib, rb, sem):
                b = base0 + j * _CHUNK
                pltpu.sync_copy(idx_hbm.at[pl.ds(b, _CHUNK)], ib)
                pltpu.async_copy(tab_hbm.at[ib], rb, sem)

            def finish(j, ib, rb, sem):
                pltpu.make_async_copy(tab_hbm.at[ib], rb, sem).wait()
                pltpu.sync_copy(rb, out_hbm.at[pl.ds(base0 + j * _CHUNK,
                                                     _CHUNK)])

            start(0, i0, r0, sem0)

            def body(i, carry):
                start(2 * i + 1, i1, r1, sem1)
                finish(2 * i, i0, r0, sem0)
                start(2 * i + 2, i0, r0, sem0)
                finish(2 * i + 1, i1, r1, sem1)
                return carry

            lax.fori_loop(0, nloop, body, 0)
            finish(full - 1, i0, r0, sem0)
            if rem:
                b = base0 + full * _CHUNK
                pltpu.sync_copy(idx_hbm.at[pl.ds(b, rem)], ir)
                pltpu.async_copy(tab_hbm.at[ir], rr, semr).wait()
                pltpu.sync_copy(rr, out_hbm.at[pl.ds(b, rem)])

        table_pass(src_hbm, gs_hbm)
        table_pass(dst_hbm, gd_hbm)

    return kern


@functools.lru_cache(None)
def _make_scatter(e, n, nh):
    per = e // _NW
    full = per // _CHUNK
    rem = per - full * _CHUNK
    rpt = ((n // 16) + 7) // 8 * 8
    rlast = n - 15 * rpt
    mesh = plsc.VectorSubcoreMesh(core_axis_name="c", subcore_axis_name="s")

    assert full % 2 == 1 and full >= 3
    nloop = (full - 1) // 2
    scratch = [pltpu.VMEM_SHARED((n, 128), f32),
               pltpu.VMEM((_CHUNK,), i32), pltpu.VMEM((_CHUNK, 128), f32),
               pltpu.VMEM((_CHUNK,), i32), pltpu.VMEM((_CHUNK, 128), f32)]
    if rem:
        scratch += [pltpu.VMEM((rem,), i32), pltpu.VMEM((rem, 128), f32)]
    scratch += [pltpu.SemaphoreType.DMA, pltpu.SemaphoreType.DMA,
                pltpu.SemaphoreType.DMA]

    @functools.partial(
        pl.kernel, mesh=mesh,
        out_type=jax.ShapeDtypeStruct((nh, 2, n, 128), f32),
        scratch_types=scratch)
    def kern(u_hbm, dst_hbm, z_hbm, out_hbm, *scr):
        if rem:
            (agg_sh, i0, u0, i1, u1, ir, ur, sem0, sem1, semr) = scr
        else:
            (agg_sh, i0, u0, i1, u1, sem0, sem1, semr) = scr
        c = lax.axis_index("c")
        s = lax.axis_index("s")
        wid = s * 2 + c
        base0 = wid * per

        def _rows(fn):
            @pl.when(s < 15)
            def _():
                fn(s * rpt, rpt)

            @pl.when(s == 15)
            def _():
                fn(15 * rpt, rlast)

        for h in range(nh):
            _rows(lambda r0, sz: pltpu.sync_copy(
                z_hbm.at[pl.ds(r0, sz)], agg_sh.at[pl.ds(r0, sz)]))
            plsc.subcore_barrier()

            def start(j, ib, ub, sem):
                b = base0 + j * _CHUNK
                pltpu.async_copy(dst_hbm.at[pl.ds(b, _CHUNK)], ib, sem)
                pltpu.async_copy(u_hbm.at[h, pl.ds(b, _CHUNK)], ub, sem)

            def finish(j, ib, ub, sem):
                b = base0 + j * _CHUNK
                pltpu.make_async_copy(dst_hbm.at[pl.ds(b, _CHUNK)], ib,
                                      sem).wait()
                pltpu.make_async_copy(u_hbm.at[h, pl.ds(b, _CHUNK)], ub,
                                      sem).wait()
                pltpu.sync_copy(ub, agg_sh.at[ib], add=True)

            start(0, i0, u0, sem0)

            def body(i, carry):
                start(2 * i + 1, i1, u1, sem1)
                finish(2 * i, i0, u0, sem0)
                start(2 * i + 2, i0, u0, sem0)
                finish(2 * i + 1, i1, u1, sem1)
                return carry

            lax.fori_loop(0, nloop, body, 0)
            finish(full - 1, i0, u0, sem0)
            if rem:
                b = base0 + full * _CHUNK
                pltpu.sync_copy(dst_hbm.at[pl.ds(b, rem)], ir)
                pltpu.sync_copy(u_hbm.at[h, pl.ds(b, rem)], ur)
                pltpu.sync_copy(ur, agg_sh.at[ir], add=True)
            plsc.subcore_barrier()
            _rows(lambda r0, sz: pltpu.sync_copy(
                agg_sh.at[pl.ds(r0, sz)], out_hbm.at[h, c, pl.ds(r0, sz)]))
            plsc.subcore_barrier()

    return kern


def _sc_gather(table, src, dst):
    n, p = table.shape
    e = src.shape[0]
    return _make_gather(e, n, p)(table, src, dst)


def _sc_scatter(u3, dst, zeros_pad):
    nh, e, _ = u3.shape
    n = zeros_pad.shape[0]
    return _make_scatter(e, n, nh)(u3, dst, zeros_pad)


def _pad_cols(x, width):
    return jnp.pad(x, ((0, 0), (0, width - x.shape[1])))


def _branch(v_in, e_attr, src, dst, emb_p, convs, dpu, npad):
    n = v_in.shape[0]
    e = e_attr.shape[0]
    d = emb_p["W"].shape[1]

    ss0, sq0 = _mm_stats(v_in, emb_p["W"], emb_p["b"][None, :], 2000)
    a0, c0 = _bn_affine(ss0, sq0, n, emb_p["g"][None, :],
                        emb_p["bt"][None, :])
    node = _emb_apply(v_in, emb_p["W"], emb_p["b"][None, :], a0, c0,
                      npad, 2000)

    zeros_pad = jnp.zeros((n, 128), f32)

    def _pad_rows(x):
        return jnp.pad(x, ((0, npad - x.shape[0]), (0, 0)))

    for p in convs:
        wm, ws = p["mlp"]["W"], p["screen"]["W"]
        wsrc = _pad_rows(jnp.concatenate(
            [_pad_cols(wm[:d], dpu), _pad_cols(ws[:d], dpu)], axis=1))
        wdst = _pad_rows(jnp.concatenate(
            [_pad_cols(wm[d:2 * d], dpu), _pad_cols(ws[d:2 * d], dpu)],
            axis=1))
        wc = jnp.pad(
            jnp.concatenate([_pad_cols(wm[2 * d:], dpu),
                             _pad_cols(ws[2 * d:], dpu)], axis=1),
            ((0, dpu - d), (0, 0)))
        bc = jnp.concatenate(
            [_pad_cols(p["mlp"]["b"][None, :], dpu),
             _pad_cols(p["screen"]["b"][None, :], dpu)], axis=1)
        pe = p["edgemlp"]
        we = _pad_cols(pe["W"], dpu)
        be = _pad_cols(pe["b"][None, :], dpu)
        sse, sqe = _mm_stats(e_attr, we, be, 4000)
        ae, ce0 = _bn_affine(sse, sqe, e, _pad_cols(pe["g"][None, :], dpu),
                             _pad_cols(pe["bt"][None, :], dpu))
        ce = ae * be + ce0

        gs, gd = _sc_gather(node, src, dst)
        ssum, ssq = _edge_stats(e_attr, gs, gd, wsrc, wdst, we, ae, ce,
                                wc, bc, 4000)
        mu = ssum / e
        var = ssq / e - mu * mu
        g_cat = jnp.concatenate(
            [_pad_cols(p["mlp"]["g"][None, :], dpu),
             _pad_cols(p["screen"]["g"][None, :], dpu)], axis=1)
        t_cat = jnp.concatenate(
            [_pad_cols(p["mlp"]["bt"][None, :], dpu),
             _pad_cols(p["screen"]["bt"][None, :], dpu)], axis=1)
        a_cat = g_cat / jnp.sqrt(var + 1e-5)
        c_cat = t_cat - a_cat * mu
        u = _edge_update(e_attr, gs, gd, wsrc, wdst, we, ae, ce, wc, bc,
                         a_cat[:, :dpu], c_cat[:, :dpu],
                         a_cat[:, dpu:], c_cat[:, dpu:], dpu, 4000)
        partials = _sc_scatter(u, dst, zeros_pad)
        s6, q6 = _agg_stats(partials, 2000)
        mu6 = s6 / n
        var6 = q6 / n - mu6 * mu6
        aa = (p["bn_g"][None, :] /
              jnp.sqrt(var6[:, :d] + 1e-5))
        ca = p["bn_b"][None, :] - aa * mu6[:, :d]
        node = _node_update(partials, node, aa, ca, d, 2000)

    return node


def kernel(va, ea, edge_index_a, node_graph_a, vs, es, edge_index_s,
           node_graph_s, params):
    n_graphs = 256
    node_a = _branch(va, ea, edge_index_a[0], edge_index_a[1],
                     params["emb_a"], params["convs_a"], 128, 128)
    node_s = _branch(vs, es, edge_index_s[0], edge_index_s[1],
                     params["emb_s"], params["convs_s"], 192, 160)

    va_sum, va_cnt = _pool_sums(node_a, node_graph_a[:, None], n_graphs, 2000)
    vs_sum, vs_cnt = _pool_sums(node_s, node_graph_s[:, None], n_graphs, 2000)

    pr = params
    plist = [pr["atten"]["W"], pr["atten"]["b"][None, :],
             pr["atten"]["g"][None, :], pr["atten"]["bt"][None, :],
             pr["atten_bn_g"][None, :], pr["atten_bn_b"][None, :]]
    for fc in pr["fcs"]:
        plist += [fc["W"], fc["b"][None, :], fc["g"][None, :],
                  fc["bt"][None, :]]
    for nm in ("pred_adsb", "pred_site"):
        plist += [pr[nm]["W"], pr[nm]["b"][None, :], pr[nm]["g"][None, :],
                  pr[nm]["bt"][None, :]]
    plist += [pr["pred_target"]["W"], pr["pred_target"]["b"][None, :]]

    return _head(va_sum, va_cnt, vs_sum, vs_cnt, 110, 150, plist)


# speedup vs baseline: 1.2066x; 1.2066x over previous
import functools

import jax
import jax.numpy as jnp
from jax import lax
from jax.experimental import pallas as pl
from jax.experimental.pallas import tpu as pltpu
from jax.experimental.pallas import tpu_sc as plsc

f32 = jnp.float32
i32 = jnp.int32

_NW = 32
_CHUNK = 128


def _sigmoid(x):
    return 1.0 / (1.0 + jnp.exp(-x))


def _silu(x):
    return x * _sigmoid(x)


def _softplus(x):
    return jnp.maximum(x, 0.0) + jnp.log(1.0 + jnp.exp(-jnp.abs(x)))


def _elu(x):
    return jnp.where(x > 0, x, jnp.exp(jnp.minimum(x, 0.0)) - 1.0)


def _softmax(x):
    e = jnp.exp(x - jnp.max(x, axis=1, keepdims=True))
    return e / jnp.sum(e, axis=1, keepdims=True)


def _bn_cols(x):
    mu = jnp.mean(x, axis=0, keepdims=True)
    var = jnp.mean((x - mu) * (x - mu), axis=0, keepdims=True)
    return (x - mu) / jnp.sqrt(var + 1e-5)


def _mm_stats(x, w, b, bn):
    n = x.shape[0]
    dout = w.shape[1]
    grid = n // bn

    def kern(x_ref, w_ref, b_ref, ss_ref, sq_ref):
        @pl.when(pl.program_id(0) == 0)
        def _():
            ss_ref[...] = jnp.zeros_like(ss_ref)
            sq_ref[...] = jnp.zeros_like(sq_ref)

        t = jnp.dot(x_ref[...], w_ref[...],
                    preferred_element_type=f32) + b_ref[...]
        ss_ref[...] += jnp.sum(t, axis=0, keepdims=True)
        sq_ref[...] += jnp.sum(t * t, axis=0, keepdims=True)

    return pl.pallas_call(
        kern, grid=(grid,),
        in_specs=[pl.BlockSpec((bn, x.shape[1]), lambda i: (i, 0)),
                  pl.BlockSpec(w.shape, lambda i: (0, 0)),
                  pl.BlockSpec(b.shape, lambda i: (0, 0))],
        out_specs=[pl.BlockSpec((1, dout), lambda i: (0, 0)),
                   pl.BlockSpec((1, dout), lambda i: (0, 0))],
        out_shape=[jax.ShapeDtypeStruct((1, dout), f32),
                   jax.ShapeDtypeStruct((1, dout), f32)])(x, w, b)


def _bn_affine(ssum, ssq, n, g, bt):
    mu = ssum / n
    var = ssq / n - mu * mu
    a = g / jnp.sqrt(var + 1e-5)
    return a, bt - a * mu


def _emb_apply(x, w, b, a, c, npad, bn):
    n = x.shape[0]
    d = w.shape[1]
    grid = n // bn

    def kern(x_ref, w_ref, b_ref, a_ref, c_ref, o_ref):
        t = jnp.dot(x_ref[...], w_ref[...],
                    preferred_element_type=f32) + b_ref[...]
        o_ref[...] = jnp.pad(_silu(a_ref[...] * t + c_ref[...]),
                             ((0, 0), (0, npad - d)))

    return pl.pallas_call(
        kern, grid=(grid,),
        in_specs=[pl.BlockSpec((bn, x.shape[1]), lambda i: (i, 0)),
                  pl.BlockSpec(w.shape, lambda i: (0, 0)),
                  pl.BlockSpec(b.shape, lambda i: (0, 0)),
                  pl.BlockSpec(a.shape, lambda i: (0, 0)),
                  pl.BlockSpec(c.shape, lambda i: (0, 0))],
        out_specs=pl.BlockSpec((bn, npad), lambda i: (i, 0)),
        out_shape=jax.ShapeDtypeStruct((n, npad), f32))(x, w, b, a, c)


def _y_block(ea, ns, nd, wsrc, wdst, we, ae, ce, wc, bc):
    t = jnp.dot(ea, we, preferred_element_type=f32)
    hm = _silu(ae * t + ce)
    return (jnp.dot(ns, wsrc, preferred_element_type=f32) +
            jnp.dot(nd, wdst, preferred_element_type=f32) +
            jnp.dot(hm, wc, preferred_element_type=f32) + bc)


def _edge_stats(ea, gs, gd, wsrc, wdst, we, ae, ce, wc, bc, bn):
    e, de = ea.shape
    npad = gs.shape[1]
    p2 = wc.shape[1]
    grid = e // bn

    def kern(ea_ref, gs_ref, gd_ref, w1_ref, w2_ref, we_ref, ae_ref, ce_ref,
             wc_ref, bc_ref, ss_ref, sq_ref):
        @pl.when(pl.program_id(0) == 0)
        def _():
            ss_ref[...] = jnp.zeros_like(ss_ref)
            sq_ref[...] = jnp.zeros_like(sq_ref)

        y = _y_block(ea_ref[...], gs_ref[...], gd_ref[...], w1_ref[...],
                     w2_ref[...], we_ref[...], ae_ref[...], ce_ref[...],
                     wc_ref[...], bc_ref[...])
        ss_ref[...] += jnp.sum(y, axis=0, keepdims=True)
        sq_ref[...] += jnp.sum(y * y, axis=0, keepdims=True)

    return pl.pallas_call(
        kern, grid=(grid,),
        in_specs=[pl.BlockSpec((bn, de), lambda i: (i, 0)),
                  pl.BlockSpec((bn, npad), lambda i: (i, 0)),
                  pl.BlockSpec((bn, npad), lambda i: (i, 0)),
                  pl.BlockSpec(wsrc.shape, lambda i: (0, 0)),
                  pl.BlockSpec(wdst.shape, lambda i: (0, 0)),
                  pl.BlockSpec(we.shape, lambda i: (0, 0)),
                  pl.BlockSpec(ae.shape, lambda i: (0, 0)),
                  pl.BlockSpec(ce.shape, lambda i: (0, 0)),
                  pl.BlockSpec(wc.shape, lambda i: (0, 0)),
                  pl.BlockSpec(bc.shape, lambda i: (0, 0))],
        out_specs=[pl.BlockSpec((1, p2), lambda i: (0, 0)),
                   pl.BlockSpec((1, p2), lambda i: (0, 0))],
        out_shape=[jax.ShapeDtypeStruct((1, p2), f32),
                   jax.ShapeDtypeStruct((1, p2), f32)])(
            ea, gs, gd, wsrc, wdst, we, ae, ce, wc, bc)


def _edge_stats_y(ea, gs, gd, wsrc, wdst, we, ae, ce, wc, bc, bn):
    e, de = ea.shape
    npad = gs.shape[1]
    p2 = wc.shape[1]
    grid = e // bn

    def kern(ea_ref, gs_ref, gd_ref, w1_ref, w2_ref, we_ref, ae_ref, ce_ref,
             wc_ref, bc_ref, ss_ref, sq_ref, y_ref):
        @pl.when(pl.program_id(0) == 0)
        def _():
            ss_ref[...] = jnp.zeros_like(ss_ref)
            sq_ref[...] = jnp.zeros_like(sq_ref)

        y = _y_block(ea_ref[...], gs_ref[...], gd_ref[...], w1_ref[...],
                     w2_ref[...], we_ref[...], ae_ref[...], ce_ref[...],
                     wc_ref[...], bc_ref[...])
        y_ref[...] = y
        ss_ref[...] += jnp.sum(y, axis=0, keepdims=True)
        sq_ref[...] += jnp.sum(y * y, axis=0, keepdims=True)

    return pl.pallas_call(
        kern, grid=(grid,),
        in_specs=[pl.BlockSpec((bn, de), lambda i: (i, 0)),
                  pl.BlockSpec((bn, npad), lambda i: (i, 0)),
                  pl.BlockSpec((bn, npad), lambda i: (i, 0)),
                  pl.BlockSpec(wsrc.shape, lambda i: (0, 0)),
                  pl.BlockSpec(wdst.shape, lambda i: (0, 0)),
                  pl.BlockSpec(we.shape, lambda i: (0, 0)),
                  pl.BlockSpec(ae.shape, lambda i: (0, 0)),
                  pl.BlockSpec(ce.shape, lambda i: (0, 0)),
                  pl.BlockSpec(wc.shape, lambda i: (0, 0)),
                  pl.BlockSpec(bc.shape, lambda i: (0, 0))],
        out_specs=[pl.BlockSpec((1, p2), lambda i: (0, 0)),
                   pl.BlockSpec((1, p2), lambda i: (0, 0)),
                   pl.BlockSpec((bn, p2), lambda i: (i, 0))],
        out_shape=[jax.ShapeDtypeStruct((1, p2), f32),
                   jax.ShapeDtypeStruct((1, p2), f32),
                   jax.ShapeDtypeStruct((e, p2), f32)])(
            ea, gs, gd, wsrc, wdst, we, ae, ce, wc, bc)


def _edge_update_y(yv, am, cm, a2, c2, dpu, bn):
    e, p2 = yv.shape
    grid = e // bn
    nh = (dpu + 127) // 128

    def kern(y_ref, am_ref, cm_ref, a2_ref, c2_ref, u_ref):
        y = y_ref[...]
        u = (_silu(am_ref[...] * y[:, :dpu] + cm_ref[...]) *
             _softplus(a2_ref[...] * y[:, dpu:] + c2_ref[...]))
        if nh * 128 > dpu:
            u = jnp.pad(u, ((0, 0), (0, nh * 128 - dpu)))
        for h in range(nh):
            u_ref[h, :, :] = u[:, h * 128:(h + 1) * 128]

    return pl.pallas_call(
        kern, grid=(grid,),
        in_specs=[pl.BlockSpec((bn, p2), lambda i: (i, 0)),
                  pl.BlockSpec(am.shape, lambda i: (0, 0)),
                  pl.BlockSpec(cm.shape, lambda i: (0, 0)),
                  pl.BlockSpec(a2.shape, lambda i: (0, 0)),
                  pl.BlockSpec(c2.shape, lambda i: (0, 0))],
        out_specs=pl.BlockSpec((nh, bn, 128), lambda i: (0, i, 0)),
        out_shape=jax.ShapeDtypeStruct((nh, e, 128), f32))(
            yv, am, cm, a2, c2)


def _edge_update(ea, gs, gd, wsrc, wdst, we, ae, ce, wc, bc, am, cm, a2, c2,
                 dpu, bn):
    e, de = ea.shape
    npad = gs.shape[1]
    grid = e // bn

    nh = (dpu + 127) // 128

    def kern(ea_ref, gs_ref, gd_ref, w1_ref, w2_ref, we_ref, ae_ref, ce_ref,
             wc_ref, bc_ref, am_ref, cm_ref, a2_ref, c2_ref, u_ref):
        y = _y_block(ea_ref[...], gs_ref[...], gd_ref[...], w1_ref[...],
                     w2_ref[...], we_ref[...], ae_ref[...], ce_ref[...],
                     wc_ref[...], bc_ref[...])
        y1 = y[:, :dpu]
        y2 = y[:, dpu:]
        u = (_silu(am_ref[...] * y1 + cm_ref[...]) *
             _softplus(a2_ref[...] * y2 + c2_ref[...]))
        if nh * 128 > dpu:
            u = jnp.pad(u, ((0, 0), (0, nh * 128 - dpu)))
        for h in range(nh):
            u_ref[h, :, :] = u[:, h * 128:(h + 1) * 128]

    return pl.pallas_call(
        kern, grid=(grid,),
        in_specs=[pl.BlockSpec((bn, de), lambda i: (i, 0)),
                  pl.BlockSpec((bn, npad), lambda i: (i, 0)),
                  pl.BlockSpec((bn, npad), lambda i: (i, 0)),
                  pl.BlockSpec(wsrc.shape, lambda i: (0, 0)),
                  pl.BlockSpec(wdst.shape, lambda i: (0, 0)),
                  pl.BlockSpec(we.shape, lambda i: (0, 0)),
                  pl.BlockSpec(ae.shape, lambda i: (0, 0)),
                  pl.BlockSpec(ce.shape, lambda i: (0, 0)),
                  pl.BlockSpec(wc.shape, lambda i: (0, 0)),
                  pl.BlockSpec(bc.shape, lambda i: (0, 0)),
                  pl.BlockSpec(am.shape, lambda i: (0, 0)),
                  pl.BlockSpec(cm.shape, lambda i: (0, 0)),
                  pl.BlockSpec(a2.shape, lambda i: (0, 0)),
                  pl.BlockSpec(c2.shape, lambda i: (0, 0))],
        out_specs=pl.BlockSpec((nh, bn, 128), lambda i: (0, i, 0)),
        out_shape=jax.ShapeDtypeStruct((nh, e, 128), f32))(
            ea, gs, gd, wsrc, wdst, we, ae, ce, wc, bc, am, cm, a2, c2)


def _agg_stats(partials, bn):
    nh, _, n, _ = partials.shape
    dpu = nh * 128
    grid = n // bn

    def kern(p_ref, ss_ref, sq_ref):
        @pl.when(pl.program_id(0) == 0)
        def _():
            ss_ref[...] = jnp.zeros_like(ss_ref)
            sq_ref[...] = jnp.zeros_like(sq_ref)

        agg = jnp.concatenate(
            [p_ref[h, 0] + p_ref[h, 1] for h in range(nh)], axis=1)
        ss_ref[...] += jnp.sum(agg, axis=0, keepdims=True)
        sq_ref[...] += jnp.sum(agg * agg, axis=0, keepdims=True)

    return pl.pallas_call(
        kern, grid=(grid,),
        in_specs=[pl.BlockSpec((nh, 2, bn, 128), lambda i: (0, 0, i, 0))],
        out_specs=[pl.BlockSpec((1, dpu), lambda i: (0, 0)),
                   pl.BlockSpec((1, dpu), lambda i: (0, 0))],
        out_shape=[jax.ShapeDtypeStruct((1, dpu), f32),
                   jax.ShapeDtypeStruct((1, dpu), f32)])(partials)


def _node_update(partials, node, aa, ca, d, bn):
    nh, _, n, _ = partials.shape
    npad = node.shape[1]
    grid = n // bn

    def kern(p_ref, nd_ref, aa_ref, ca_ref, o_ref):
        agg = jnp.concatenate(
            [p_ref[h, 0] + p_ref[h, 1] for h in range(nh)], axis=1)[:, :d]
        res = _softplus(aa_ref[...] * agg + ca_ref[...] + nd_ref[:, :d])
        o_ref[...] = jnp.pad(res, ((0, 0), (0, npad - d)))

    return pl.pallas_call(
        kern, grid=(grid,),
        in_specs=[pl.BlockSpec((nh, 2, bn, 128), lambda i: (0, 0, i, 0)),
                  pl.BlockSpec((bn, npad), lambda i: (i, 0)),
                  pl.BlockSpec(aa.shape, lambda i: (0, 0)),
                  pl.BlockSpec(ca.shape, lambda i: (0, 0))],
        out_specs=pl.BlockSpec((bn, npad), lambda i: (i, 0)),
        out_shape=jax.ShapeDtypeStruct((n, npad), f32))(partials, node, aa, ca)


def _pool_sums(node, gid2d, n_graphs, bn):
    n, d = node.shape
    grid = n // bn

    def kern(g_ref, v_ref, s_ref, c_ref):
        @pl.when(pl.program_id(0) == 0)
        def _():
            s_ref[...] = jnp.zeros_like(s_ref)
            c_ref[...] = jnp.zeros_like(c_ref)

        gid = g_ref[...]
        oh = (gid == lax.broadcasted_iota(i32, (bn, n_graphs), 1)).astype(f32)
        s_ref[...] += lax.dot_general(oh, v_ref[...], (((0,), (0,)), ((), ())),
                                      preferred_element_type=f32,
                                      precision=lax.Precision.HIGHEST)
        c_ref[...] += lax.dot_general(
            oh, jnp.ones((bn, 1), f32), (((0,), (0,)), ((), ())),
            preferred_element_type=f32, precision=lax.Precision.HIGHEST)

    return pl.pallas_call(
        kern, grid=(grid,),
        in_specs=[pl.BlockSpec((bn, 1), lambda i: (i, 0)),
                  pl.BlockSpec((bn, d), lambda i: (i, 0))],
        out_specs=[pl.BlockSpec((n_graphs, d), lambda i: (0, 0)),
                   pl.BlockSpec((n_graphs, 1), lambda i: (0, 0))],
        out_shape=[jax.ShapeDtypeStruct((n_graphs, d), f32),
                   jax.ShapeDtypeStruct((n_graphs, 1), f32)])(gid2d, node)


def _head(va_sum, va_cnt, vs_sum, vs_cnt, da, ds, plist):
    g = va_sum.shape[0]

    def kern(vas_ref, vac_ref, vss_ref, vsc_ref,
             atw_ref, atb_ref, atg_ref, att_ref, abg_ref, abb_ref,
             f0w_ref, f0b_ref, f0g_ref, f0t_ref,
             f1w_ref, f1b_ref, f1g_ref, f1t_ref,
             f2w_ref, f2b_ref, f2g_ref, f2t_ref,
             paw_ref, pab_ref, pag_ref, pat_ref,
             psw_ref, psb_ref, psg_ref, pst_ref,
             ptw_ref, ptb_ref, o_ref):
        va_s = vas_ref[:, :da] / jnp.maximum(vac_ref[...], 1.0)
        vs_s = vss_ref[:, :ds] / jnp.maximum(vsc_ref[...], 1.0)
        vt = jnp.concatenate([va_s, vs_s], axis=1)
        h = jnp.dot(vt, atw_ref[...], preferred_element_type=f32) + atb_ref[...]
        h = _elu(atg_ref[...] * _bn_cols(h) + att_ref[...]) * vt
        vt2 = abg_ref[...] * _bn_cols(h) + abb_ref[...]
        x = vt2
        for ww, bb, gg, tt in ((f0w_ref, f0b_ref, f0g_ref, f0t_ref),
                               (f1w_ref, f1b_ref, f1g_ref, f1t_ref),
                               (f2w_ref, f2b_ref, f2g_ref, f2t_ref)):
            x = jnp.dot(x, ww[...], preferred_element_type=f32) + bb[...]
            x = _silu(gg[...] * _bn_cols(x) + tt[...])
        ya = jnp.dot(x, paw_ref[...], preferred_element_type=f32) + pab_ref[...]
        ca = _softmax(pag_ref[...] * _bn_cols(ya) + pat_ref[...])
        ys = jnp.dot(x, psw_ref[...], preferred_element_type=f32) + psb_ref[...]
        cs = _softmax(psg_ref[...] * _bn_cols(ys) + pst_ref[...])
        t = jnp.dot(x, ptw_ref[...], preferred_element_type=f32) + ptb_ref[...]
        o_ref[...] = jnp.concatenate([ca, cs, t], axis=1)

    args = [va_sum, va_cnt, vs_sum, vs_cnt] + plist
    return pl.pallas_call(
        kern,
        in_specs=[pl.BlockSpec(a.shape, None) for a in args],
        out_specs=pl.BlockSpec((g, 10), None),
        out_shape=jax.ShapeDtypeStruct((g, 10), f32))(*args)


@functools.lru_cache(None)
def _make_gather(e, n, p):
    per = e // _NW
    full = per // _CHUNK
    rem = per - full * _CHUNK
    assert full % 2 == 1 and full >= 3
    nloop = (full - 1) // 2
    mesh = plsc.VectorSubcoreMesh(core_axis_name="c", subcore_axis_name="s")

    scratch = [pltpu.VMEM((_CHUNK,), i32), pltpu.VMEM((_CHUNK, p), f32),
               pltpu.VMEM((_CHUNK,), i32), pltpu.VMEM((_CHUNK, p), f32)]
    if rem:
        scratch += [pltpu.VMEM((rem,), i32), pltpu.VMEM((rem, p), f32)]
    scratch += [pltpu.SemaphoreType.DMA, pltpu.SemaphoreType.DMA,
                pltpu.SemaphoreType.DMA]

    @functools.partial(
        pl.kernel, mesh=mesh,
        out_type=[jax.ShapeDtypeStruct((e, p), f32),
                  jax.ShapeDtypeStruct((e, p), f32)],
        scratch_types=scratch)
    def kern(tab_hbm, src_hbm, dst_hbm, gs_hbm, gd_hbm, *scr):
        if rem:
            (i0, r0, i1, r1, ir, rr, sem0, sem1, semr) = scr
        else:
            (i0, r0, i1, r1, sem0, sem1, semr) = scr
        wid = lax.axis_index("s") * 2 + lax.axis_index("c")
        base0 = wid * per

        def table_pass(idx_hbm, out_hbm):
            def start(j, ib, rb, sem):
                b = base0 + j * _CHUNK
                pltpu.sync_copy(idx_hbm.at[pl.ds(b, _CHUNK)], ib)
                pltpu.async_copy(tab_hbm.at[ib], rb, sem)

            def finish(j, ib, rb, sem):
                pltpu.make_async_copy(tab_hbm.at[ib], rb, sem).wait()
                pltpu.sync_copy(rb, out_hbm.at[pl.ds(base0 + j * _CHUNK,
                                                     _CHUNK)])

            start(0, i0, r0, sem0)

            def body(i, carry):
                start(2 * i + 1, i1, r1, sem1)
                finish(2 * i, i0, r0, sem0)
                start(2 * i + 2, i0, r0, sem0)
                finish(2 * i + 1, i1, r1, sem1)
                return carry

            lax.fori_loop(0, nloop, body, 0)
            finish(full - 1, i0, r0, sem0)
            if rem:
                b = base0 + full * _CHUNK
                pltpu.sync_copy(idx_hbm.at[pl.ds(b, rem)], ir)
                pltpu.async_copy(tab_hbm.at[ir], rr, semr).wait()
                pltpu.sync_copy(rr, out_hbm.at[pl.ds(b, rem)])

        table_pass(src_hbm, gs_hbm)
        table_pass(dst_hbm, gd_hbm)

    return kern


@functools.lru_cache(None)
def _make_scatter(e, n, nh):
    per = e // _NW
    full = per // _CHUNK
    rem = per - full * _CHUNK
    rpt = ((n // 16) + 7) // 8 * 8
    rlast = n - 15 * rpt
    mesh = plsc.VectorSubcoreMesh(core_axis_name="c", subcore_axis_name="s")

    assert full % 2 == 1 and full >= 3
    nloop = (full - 1) // 2
    scratch = [pltpu.VMEM_SHARED((n, 128), f32),
               pltpu.VMEM((_CHUNK,), i32), pltpu.VMEM((_CHUNK, 128), f32),
               pltpu.VMEM((_CHUNK,), i32), pltpu.VMEM((_CHUNK, 128), f32)]
    if rem:
        scratch += [pltpu.VMEM((rem,), i32), pltpu.VMEM((rem, 128), f32)]
    scratch += [pltpu.SemaphoreType.DMA, pltpu.SemaphoreType.DMA,
                pltpu.SemaphoreType.DMA]

    @functools.partial(
        pl.kernel, mesh=mesh,
        out_type=jax.ShapeDtypeStruct((nh, 2, n, 128), f32),
        scratch_types=scratch)
    def kern(u_hbm, dst_hbm, z_hbm, out_hbm, *scr):
        if rem:
            (agg_sh, i0, u0, i1, u1, ir, ur, sem0, sem1, semr) = scr
        else:
            (agg_sh, i0, u0, i1, u1, sem0, sem1, semr) = scr
        c = lax.axis_index("c")
        s = lax.axis_index("s")
        wid = s * 2 + c
        base0 = wid * per

        def _rows(fn):
            @pl.when(s < 15)
            def _():
                fn(s * rpt, rpt)

            @pl.when(s == 15)
            def _():
                fn(15 * rpt, rlast)

        for h in range(nh):
            _rows(lambda r0, sz: pltpu.sync_copy(
                z_hbm.at[pl.ds(r0, sz)], agg_sh.at[pl.ds(r0, sz)]))
            plsc.subcore_barrier()

            def start(j, ib, ub, sem):
                b = base0 + j * _CHUNK
                pltpu.async_copy(dst_hbm.at[pl.ds(b, _CHUNK)], ib, sem)
                pltpu.async_copy(u_hbm.at[h, pl.ds(b, _CHUNK)], ub, sem)

            def finish(j, ib, ub, sem):
                b = base0 + j * _CHUNK
                pltpu.make_async_copy(dst_hbm.at[pl.ds(b, _CHUNK)], ib,
                                      sem).wait()
                pltpu.make_async_copy(u_hbm.at[h, pl.ds(b, _CHUNK)], ub,
                                      sem).wait()
                pltpu.sync_copy(ub, agg_sh.at[ib], add=True)

            start(0, i0, u0, sem0)

            def body(i, carry):
                start(2 * i + 1, i1, u1, sem1)
                finish(2 * i, i0, u0, sem0)
                start(2 * i + 2, i0, u0, sem0)
                finish(2 * i + 1, i1, u1, sem1)
                return carry

            lax.fori_loop(0, nloop, body, 0)
            finish(full - 1, i0, u0, sem0)
            if rem:
                b = base0 + full * _CHUNK
                pltpu.sync_copy(dst_hbm.at[pl.ds(b, rem)], ir)
                pltpu.sync_copy(u_hbm.at[h, pl.ds(b, rem)], ur)
                pltpu.sync_copy(ur, agg_sh.at[ir], add=True)
            plsc.subcore_barrier()
            _rows(lambda r0, sz: pltpu.sync_copy(
                agg_sh.at[pl.ds(r0, sz)], out_hbm.at[h, c, pl.ds(r0, sz)]))
            plsc.subcore_barrier()

    return kern


def _sc_gather(table, src, dst):
    n, p = table.shape
    e = src.shape[0]
    return _make_gather(e, n, p)(table, src, dst)


def _sc_scatter(u3, dst, zeros_pad):
    nh, e, _ = u3.shape
    n = zeros_pad.shape[0]
    return _make_scatter(e, n, nh)(u3, dst, zeros_pad)


def _pad_cols(x, width):
    return jnp.pad(x, ((0, 0), (0, width - x.shape[1])))


def _branch(v_in, e_attr, src, dst, emb_p, convs, dpu, npad, store_y):
    n = v_in.shape[0]
    e = e_attr.shape[0]
    d = emb_p["W"].shape[1]

    ss0, sq0 = _mm_stats(v_in, emb_p["W"], emb_p["b"][None, :], 2000)
    a0, c0 = _bn_affine(ss0, sq0, n, emb_p["g"][None, :],
                        emb_p["bt"][None, :])
    node = _emb_apply(v_in, emb_p["W"], emb_p["b"][None, :], a0, c0,
                      npad, 2000)

    zeros_pad = jnp.zeros((n, 128), f32)

    def _pad_rows(x):
        return jnp.pad(x, ((0, npad - x.shape[0]), (0, 0)))

    for p in convs:
        wm, ws = p["mlp"]["W"], p["screen"]["W"]
        wsrc = _pad_rows(jnp.concatenate(
            [_pad_cols(wm[:d], dpu), _pad_cols(ws[:d], dpu)], axis=1))
        wdst = _pad_rows(jnp.concatenate(
            [_pad_cols(wm[d:2 * d], dpu), _pad_cols(ws[d:2 * d], dpu)],
            axis=1))
        wc = jnp.pad(
            jnp.concatenate([_pad_cols(wm[2 * d:], dpu),
                             _pad_cols(ws[2 * d:], dpu)], axis=1),
            ((0, dpu - d), (0, 0)))
        bc = jnp.concatenate(
            [_pad_cols(p["mlp"]["b"][None, :], dpu),
             _pad_cols(p["screen"]["b"][None, :], dpu)], axis=1)
        pe = p["edgemlp"]
        we = _pad_cols(pe["W"], dpu)
        be = _pad_cols(pe["b"][None, :], dpu)
        sse, sqe = _mm_stats(e_attr, we, be, 4000)
        ae, ce0 = _bn_affine(sse, sqe, e, _pad_cols(pe["g"][None, :], dpu),
                             _pad_cols(pe["bt"][None, :], dpu))
        ce = ae * be + ce0

        gs, gd = _sc_gather(node, src, dst)
        if store_y:
            ssum, ssq, yv = _edge_stats_y(e_attr, gs, gd, wsrc, wdst, we,
                                          ae, ce, wc, bc, 4000)
        else:
            ssum, ssq = _edge_stats(e_attr, gs, gd, wsrc, wdst, we, ae, ce,
                                    wc, bc, 4000)
        mu = ssum / e
        var = ssq / e - mu * mu
        g_cat = jnp.concatenate(
            [_pad_cols(p["mlp"]["g"][None, :], dpu),
             _pad_cols(p["screen"]["g"][None, :], dpu)], axis=1)
        t_cat = jnp.concatenate(
            [_pad_cols(p["mlp"]["bt"][None, :], dpu),
             _pad_cols(p["screen"]["bt"][None, :], dpu)], axis=1)
        a_cat = g_cat / jnp.sqrt(var + 1e-5)
        c_cat = t_cat - a_cat * mu
        if store_y:
            u = _edge_update_y(yv, a_cat[:, :dpu], c_cat[:, :dpu],
                               a_cat[:, dpu:], c_cat[:, dpu:], dpu, 4000)
        else:
            u = _edge_update(e_attr, gs, gd, wsrc, wdst, we, ae, ce, wc, bc,
                             a_cat[:, :dpu], c_cat[:, :dpu],
                             a_cat[:, dpu:], c_cat[:, dpu:], dpu, 4000)
        partials = _sc_scatter(u, dst, zeros_pad)
        s6, q6 = _agg_stats(partials, 2000)
        mu6 = s6 / n
        var6 = q6 / n - mu6 * mu6
        aa = (p["bn_g"][None, :] /
              jnp.sqrt(var6[:, :d] + 1e-5))
        ca = p["bn_b"][None, :] - aa * mu6[:, :d]
        node = _node_update(partials, node, aa, ca, d, 2000)

    return node


def kernel(va, ea, edge_index_a, node_graph_a, vs, es, edge_index_s,
           node_graph_s, params):
    n_graphs = 256
    node_a = _branch(va, ea, edge_index_a[0], edge_index_a[1],
                     params["emb_a"], params["convs_a"], 128, 128, False)
    node_s = _branch(vs, es, edge_index_s[0], edge_index_s[1],
                     params["emb_s"], params["convs_s"], 192, 256, True)

    va_sum, va_cnt = _pool_sums(node_a, node_graph_a[:, None], n_graphs, 2000)
    vs_sum, vs_cnt = _pool_sums(node_s, node_graph_s[:, None], n_graphs, 2000)

    pr = params
    plist = [pr["atten"]["W"], pr["atten"]["b"][None, :],
             pr["atten"]["g"][None, :], pr["atten"]["bt"][None, :],
             pr["atten_bn_g"][None, :], pr["atten_bn_b"][None, :]]
    for fc in pr["fcs"]:
        plist += [fc["W"], fc["b"][None, :], fc["g"][None, :],
                  fc["bt"][None, :]]
    for nm in ("pred_adsb", "pred_site"):
        plist += [pr[nm]["W"], pr[nm]["b"][None, :], pr[nm]["g"][None, :],
                  pr[nm]["bt"][None, :]]
    plist += [pr["pred_target"]["W"], pr["pred_target"]["b"][None, :]]

    return _head(va_sum, va_cnt, vs_sum, vs_cnt, 110, 150, plist)


# speedup vs baseline: 1.2138x; 1.0060x over previous
import functools

import jax
import jax.numpy as jnp
from jax import lax
from jax.experimental import pallas as pl
from jax.experimental.pallas import tpu as pltpu
from jax.experimental.pallas import tpu_sc as plsc

f32 = jnp.float32
i32 = jnp.int32

_NW = 32
_CHUNK = 128


def _sigmoid(x):
    return 1.0 / (1.0 + jnp.exp(-x))


def _silu(x):
    return x * _sigmoid(x)


def _softplus(x):
    return jnp.maximum(x, 0.0) + jnp.log(1.0 + jnp.exp(-jnp.abs(x)))


def _elu(x):
    return jnp.where(x > 0, x, jnp.exp(jnp.minimum(x, 0.0)) - 1.0)


def _softmax(x):
    e = jnp.exp(x - jnp.max(x, axis=1, keepdims=True))
    return e / jnp.sum(e, axis=1, keepdims=True)


def _bn_cols(x):
    mu = jnp.mean(x, axis=0, keepdims=True)
    var = jnp.mean((x - mu) * (x - mu), axis=0, keepdims=True)
    return (x - mu) / jnp.sqrt(var + 1e-5)


def _mm_stats(x, w, b, bn):
    n = x.shape[0]
    dout = w.shape[1]
    grid = n // bn

    def kern(x_ref, w_ref, b_ref, ss_ref, sq_ref):
        @pl.when(pl.program_id(0) == 0)
        def _():
            ss_ref[...] = jnp.zeros_like(ss_ref)
            sq_ref[...] = jnp.zeros_like(sq_ref)

        t = jnp.dot(x_ref[...], w_ref[...],
                    preferred_element_type=f32) + b_ref[...]
        ss_ref[...] += jnp.sum(t, axis=0, keepdims=True)
        sq_ref[...] += jnp.sum(t * t, axis=0, keepdims=True)

    return pl.pallas_call(
        kern, grid=(grid,),
        in_specs=[pl.BlockSpec((bn, x.shape[1]), lambda i: (i, 0)),
                  pl.BlockSpec(w.shape, lambda i: (0, 0)),
                  pl.BlockSpec(b.shape, lambda i: (0, 0))],
        out_specs=[pl.BlockSpec((1, dout), lambda i: (0, 0)),
                   pl.BlockSpec((1, dout), lambda i: (0, 0))],
        out_shape=[jax.ShapeDtypeStruct((1, dout), f32),
                   jax.ShapeDtypeStruct((1, dout), f32)])(x, w, b)


def _bn_affine(ssum, ssq, n, g, bt):
    mu = ssum / n
    var = ssq / n - mu * mu
    a = g / jnp.sqrt(var + 1e-5)
    return a, bt - a * mu


def _emb_apply(x, w, b, a, c, npad, bn):
    n = x.shape[0]
    d = w.shape[1]
    grid = n // bn

    def kern(x_ref, w_ref, b_ref, a_ref, c_ref, o_ref):
        t = jnp.dot(x_ref[...], w_ref[...],
                    preferred_element_type=f32) + b_ref[...]
        o_ref[...] = jnp.pad(_silu(a_ref[...] * t + c_ref[...]),
                             ((0, 0), (0, npad - d)))

    return pl.pallas_call(
        kern, grid=(grid,),
        in_specs=[pl.BlockSpec((bn, x.shape[1]), lambda i: (i, 0)),
                  pl.BlockSpec(w.shape, lambda i: (0, 0)),
                  pl.BlockSpec(b.shape, lambda i: (0, 0)),
                  pl.BlockSpec(a.shape, lambda i: (0, 0)),
                  pl.BlockSpec(c.shape, lambda i: (0, 0))],
        out_specs=pl.BlockSpec((bn, npad), lambda i: (i, 0)),
        out_shape=jax.ShapeDtypeStruct((n, npad), f32))(x, w, b, a, c)


def _y_block(ea, ns, nd, wsrc, wdst, we, ae, ce, wc, bc):
    t = jnp.dot(ea, we, preferred_element_type=f32)
    hm = _silu(ae * t + ce)
    return (jnp.dot(ns, wsrc, preferred_element_type=f32) +
            jnp.dot(nd, wdst, preferred_element_type=f32) +
            jnp.dot(hm, wc, preferred_element_type=f32) + bc)


def _edge_stats(ea, gs, gd, wsrc, wdst, we, ae, ce, wc, bc, bn):
    e, de = ea.shape
    npad = gs.shape[1]
    p2 = wc.shape[1]
    grid = e // bn

    def kern(ea_ref, gs_ref, gd_ref, w1_ref, w2_ref, we_ref, ae_ref, ce_ref,
             wc_ref, bc_ref, ss_ref, sq_ref):
        @pl.when(pl.program_id(0) == 0)
        def _():
            ss_ref[...] = jnp.zeros_like(ss_ref)
            sq_ref[...] = jnp.zeros_like(sq_ref)

        y = _y_block(ea_ref[...], gs_ref[...], gd_ref[...], w1_ref[...],
                     w2_ref[...], we_ref[...], ae_ref[...], ce_ref[...],
                     wc_ref[...], bc_ref[...])
        ss_ref[...] += jnp.sum(y, axis=0, keepdims=True)
        sq_ref[...] += jnp.sum(y * y, axis=0, keepdims=True)

    return pl.pallas_call(
        kern, grid=(grid,),
        in_specs=[pl.BlockSpec((bn, de), lambda i: (i, 0)),
                  pl.BlockSpec((bn, npad), lambda i: (i, 0)),
                  pl.BlockSpec((bn, npad), lambda i: (i, 0)),
                  pl.BlockSpec(wsrc.shape, lambda i: (0, 0)),
                  pl.BlockSpec(wdst.shape, lambda i: (0, 0)),
                  pl.BlockSpec(we.shape, lambda i: (0, 0)),
                  pl.BlockSpec(ae.shape, lambda i: (0, 0)),
                  pl.BlockSpec(ce.shape, lambda i: (0, 0)),
                  pl.BlockSpec(wc.shape, lambda i: (0, 0)),
                  pl.BlockSpec(bc.shape, lambda i: (0, 0))],
        out_specs=[pl.BlockSpec((1, p2), lambda i: (0, 0)),
                   pl.BlockSpec((1, p2), lambda i: (0, 0))],
        out_shape=[jax.ShapeDtypeStruct((1, p2), f32),
                   jax.ShapeDtypeStruct((1, p2), f32)])(
            ea, gs, gd, wsrc, wdst, we, ae, ce, wc, bc)


def _edge_stats_y(ea, gs, gd, wsrc, wdst, we, ae, ce, wc, bc, bn):
    e, de = ea.shape
    npad = gs.shape[1]
    p2 = wc.shape[1]
    grid = e // bn

    def kern(ea_ref, gs_ref, gd_ref, w1_ref, w2_ref, we_ref, ae_ref, ce_ref,
             wc_ref, bc_ref, ss_ref, sq_ref, y_ref):
        @pl.when(pl.program_id(0) == 0)
        def _():
            ss_ref[...] = jnp.zeros_like(ss_ref)
            sq_ref[...] = jnp.zeros_like(sq_ref)

        y = _y_block(ea_ref[...], gs_ref[...], gd_ref[...], w1_ref[...],
                     w2_ref[...], we_ref[...], ae_ref[...], ce_ref[...],
                     wc_ref[...], bc_ref[...])
        y_ref[...] = y
        ss_ref[...] += jnp.sum(y, axis=0, keepdims=True)
        sq_ref[...] += jnp.sum(y * y, axis=0, keepdims=True)

    return pl.pallas_call(
        kern, grid=(grid,),
        in_specs=[pl.BlockSpec((bn, de), lambda i: (i, 0)),
                  pl.BlockSpec((bn, npad), lambda i: (i, 0)),
                  pl.BlockSpec((bn, npad), lambda i: (i, 0)),
                  pl.BlockSpec(wsrc.shape, lambda i: (0, 0)),
                  pl.BlockSpec(wdst.shape, lambda i: (0, 0)),
                  pl.BlockSpec(we.shape, lambda i: (0, 0)),
                  pl.BlockSpec(ae.shape, lambda i: (0, 0)),
                  pl.BlockSpec(ce.shape, lambda i: (0, 0)),
                  pl.BlockSpec(wc.shape, lambda i: (0, 0)),
                  pl.BlockSpec(bc.shape, lambda i: (0, 0))],
        out_specs=[pl.BlockSpec((1, p2), lambda i: (0, 0)),
                   pl.BlockSpec((1, p2), lambda i: (0, 0)),
                   pl.BlockSpec((bn, p2), lambda i: (i, 0))],
        out_shape=[jax.ShapeDtypeStruct((1, p2), f32),
                   jax.ShapeDtypeStruct((1, p2), f32),
                   jax.ShapeDtypeStruct((e, p2), f32)])(
            ea, gs, gd, wsrc, wdst, we, ae, ce, wc, bc)


def _edge_update_y(yv, am, cm, a2, c2, dpu, bn):
    e, p2 = yv.shape
    grid = e // bn
    nh = (dpu + 127) // 128

    def kern(y_ref, am_ref, cm_ref, a2_ref, c2_ref, u_ref):
        y = y_ref[...]
        u = (_silu(am_ref[...] * y[:, :dpu] + cm_ref[...]) *
             _softplus(a2_ref[...] * y[:, dpu:] + c2_ref[...]))
        if nh * 128 > dpu:
            u = jnp.pad(u, ((0, 0), (0, nh * 128 - dpu)))
        for h in range(nh):
            u_ref[h, :, :] = u[:, h * 128:(h + 1) * 128]

    return pl.pallas_call(
        kern, grid=(grid,),
        in_specs=[pl.BlockSpec((bn, p2), lambda i: (i, 0)),
                  pl.BlockSpec(am.shape, lambda i: (0, 0)),
                  pl.BlockSpec(cm.shape, lambda i: (0, 0)),
                  pl.BlockSpec(a2.shape, lambda i: (0, 0)),
                  pl.BlockSpec(c2.shape, lambda i: (0, 0))],
        out_specs=pl.BlockSpec((nh, bn, 128), lambda i: (0, i, 0)),
        out_shape=jax.ShapeDtypeStruct((nh, e, 128), f32))(
            yv, am, cm, a2, c2)


def _edge_update(ea, gs, gd, wsrc, wdst, we, ae, ce, wc, bc, am, cm, a2, c2,
                 dpu, bn):
    e, de = ea.shape
    npad = gs.shape[1]
    grid = e // bn

    nh = (dpu + 127) // 128

    def kern(ea_ref, gs_ref, gd_ref, w1_ref, w2_ref, we_ref, ae_ref, ce_ref,
             wc_ref, bc_ref, am_ref, cm_ref, a2_ref, c2_ref, u_ref):
        y = _y_block(ea_ref[...], gs_ref[...], gd_ref[...], w1_ref[...],
                     w2_ref[...], we_ref[...], ae_ref[...], ce_ref[...],
                     wc_ref[...], bc_ref[...])
        y1 = y[:, :dpu]
        y2 = y[:, dpu:]
        u = (_silu(am_ref[...] * y1 + cm_ref[...]) *
             _softplus(a2_ref[...] * y2 + c2_ref[...]))
        if nh * 128 > dpu:
            u = jnp.pad(u, ((0, 0), (0, nh * 128 - dpu)))
        for h in range(nh):
            u_ref[h, :, :] = u[:, h * 128:(h + 1) * 128]

    return pl.pallas_call(
        kern, grid=(grid,),
        in_specs=[pl.BlockSpec((bn, de), lambda i: (i, 0)),
                  pl.BlockSpec((bn, npad), lambda i: (i, 0)),
                  pl.BlockSpec((bn, npad), lambda i: (i, 0)),
                  pl.BlockSpec(wsrc.shape, lambda i: (0, 0)),
                  pl.BlockSpec(wdst.shape, lambda i: (0, 0)),
                  pl.BlockSpec(we.shape, lambda i: (0, 0)),
                  pl.BlockSpec(ae.shape, lambda i: (0, 0)),
                  pl.BlockSpec(ce.shape, lambda i: (0, 0)),
                  pl.BlockSpec(wc.shape, lambda i: (0, 0)),
                  pl.BlockSpec(bc.shape, lambda i: (0, 0)),
                  pl.BlockSpec(am.shape, lambda i: (0, 0)),
                  pl.BlockSpec(cm.shape, lambda i: (0, 0)),
                  pl.BlockSpec(a2.shape, lambda i: (0, 0)),
                  pl.BlockSpec(c2.shape, lambda i: (0, 0))],
        out_specs=pl.BlockSpec((nh, bn, 128), lambda i: (0, i, 0)),
        out_shape=jax.ShapeDtypeStruct((nh, e, 128), f32))(
            ea, gs, gd, wsrc, wdst, we, ae, ce, wc, bc, am, cm, a2, c2)


def _agg_stats(partials, bn):
    nh, _, n, _ = partials.shape
    dpu = nh * 128
    grid = n // bn

    def kern(p_ref, ss_ref, sq_ref):
        @pl.when(pl.program_id(0) == 0)
        def _():
            ss_ref[...] = jnp.zeros_like(ss_ref)
            sq_ref[...] = jnp.zeros_like(sq_ref)

        agg = jnp.concatenate(
            [p_ref[h, 0] + p_ref[h, 1] for h in range(nh)], axis=1)
        ss_ref[...] += jnp.sum(agg, axis=0, keepdims=True)
        sq_ref[...] += jnp.sum(agg * agg, axis=0, keepdims=True)

    return pl.pallas_call(
        kern, grid=(grid,),
        in_specs=[pl.BlockSpec((nh, 2, bn, 128), lambda i: (0, 0, i, 0))],
        out_specs=[pl.BlockSpec((1, dpu), lambda i: (0, 0)),
                   pl.BlockSpec((1, dpu), lambda i: (0, 0))],
        out_shape=[jax.ShapeDtypeStruct((1, dpu), f32),
                   jax.ShapeDtypeStruct((1, dpu), f32)])(partials)


def _node_update(partials, node, aa, ca, d, bn):
    nh, _, n, _ = partials.shape
    npad = node.shape[1]
    grid = n // bn

    def kern(p_ref, nd_ref, aa_ref, ca_ref, o_ref):
        agg = jnp.concatenate(
            [p_ref[h, 0] + p_ref[h, 1] for h in range(nh)], axis=1)[:, :d]
        res = _softplus(aa_ref[...] * agg + ca_ref[...] + nd_ref[:, :d])
        o_ref[...] = jnp.pad(res, ((0, 0), (0, npad - d)))

    return pl.pallas_call(
        kern, grid=(grid,),
        in_specs=[pl.BlockSpec((nh, 2, bn, 128), lambda i: (0, 0, i, 0)),
                  pl.BlockSpec((bn, npad), lambda i: (i, 0)),
                  pl.BlockSpec(aa.shape, lambda i: (0, 0)),
                  pl.BlockSpec(ca.shape, lambda i: (0, 0))],
        out_specs=pl.BlockSpec((bn, npad), lambda i: (i, 0)),
        out_shape=jax.ShapeDtypeStruct((n, npad), f32))(partials, node, aa, ca)


def _pool_sums(node, gid2d, n_graphs, bn):
    n, d = node.shape
    grid = n // bn

    def kern(g_ref, v_ref, s_ref, c_ref):
        @pl.when(pl.program_id(0) == 0)
        def _():
            s_ref[...] = jnp.zeros_like(s_ref)
            c_ref[...] = jnp.zeros_like(c_ref)

        gid = g_ref[...]
        oh = (gid == lax.broadcasted_iota(i32, (bn, n_graphs), 1)).astype(f32)
        s_ref[...] += lax.dot_general(oh, v_ref[...], (((0,), (0,)), ((), ())),
                                      preferred_element_type=f32,
                                      precision=lax.Precision.HIGHEST)
        c_ref[...] += lax.dot_general(
            oh, jnp.ones((bn, 1), f32), (((0,), (0,)), ((), ())),
            preferred_element_type=f32, precision=lax.Precision.HIGHEST)

    return pl.pallas_call(
        kern, grid=(grid,),
        in_specs=[pl.BlockSpec((bn, 1), lambda i: (i, 0)),
                  pl.BlockSpec((bn, d), lambda i: (i, 0))],
        out_specs=[pl.BlockSpec((n_graphs, d), lambda i: (0, 0)),
                   pl.BlockSpec((n_graphs, 1), lambda i: (0, 0))],
        out_shape=[jax.ShapeDtypeStruct((n_graphs, d), f32),
                   jax.ShapeDtypeStruct((n_graphs, 1), f32)])(gid2d, node)


def _head(va_sum, va_cnt, vs_sum, vs_cnt, da, ds, plist):
    g = va_sum.shape[0]

    def kern(vas_ref, vac_ref, vss_ref, vsc_ref,
             atw_ref, atb_ref, atg_ref, att_ref, abg_ref, abb_ref,
             f0w_ref, f0b_ref, f0g_ref, f0t_ref,
             f1w_ref, f1b_ref, f1g_ref, f1t_ref,
             f2w_ref, f2b_ref, f2g_ref, f2t_ref,
             paw_ref, pab_ref, pag_ref, pat_ref,
             psw_ref, psb_ref, psg_ref, pst_ref,
             ptw_ref, ptb_ref, o_ref):
        va_s = vas_ref[:, :da] / jnp.maximum(vac_ref[...], 1.0)
        vs_s = vss_ref[:, :ds] / jnp.maximum(vsc_ref[...], 1.0)
        vt = jnp.concatenate([va_s, vs_s], axis=1)
        h = jnp.dot(vt, atw_ref[...], preferred_element_type=f32) + atb_ref[...]
        h = _elu(atg_ref[...] * _bn_cols(h) + att_ref[...]) * vt
        vt2 = abg_ref[...] * _bn_cols(h) + abb_ref[...]
        x = vt2
        for ww, bb, gg, tt in ((f0w_ref, f0b_ref, f0g_ref, f0t_ref),
                               (f1w_ref, f1b_ref, f1g_ref, f1t_ref),
                               (f2w_ref, f2b_ref, f2g_ref, f2t_ref)):
            x = jnp.dot(x, ww[...], preferred_element_type=f32) + bb[...]
            x = _silu(gg[...] * _bn_cols(x) + tt[...])
        ya = jnp.dot(x, paw_ref[...], preferred_element_type=f32) + pab_ref[...]
        ca = _softmax(pag_ref[...] * _bn_cols(ya) + pat_ref[...])
        ys = jnp.dot(x, psw_ref[...], preferred_element_type=f32) + psb_ref[...]
        cs = _softmax(psg_ref[...] * _bn_cols(ys) + pst_ref[...])
        t = jnp.dot(x, ptw_ref[...], preferred_element_type=f32) + ptb_ref[...]
        o_ref[...] = jnp.concatenate([ca, cs, t], axis=1)

    args = [va_sum, va_cnt, vs_sum, vs_cnt] + plist
    return pl.pallas_call(
        kern,
        in_specs=[pl.BlockSpec(a.shape, None) for a in args],
        out_specs=pl.BlockSpec((g, 10), None),
        out_shape=jax.ShapeDtypeStruct((g, 10), f32))(*args)


@functools.lru_cache(None)
def _make_gather(e, n, p):
    per = e // _NW
    full = per // _CHUNK
    rem = per - full * _CHUNK
    assert full % 2 == 1 and full >= 3
    nloop = (full - 1) // 2
    mesh = plsc.VectorSubcoreMesh(core_axis_name="c", subcore_axis_name="s")

    scratch = [pltpu.VMEM((_CHUNK,), i32), pltpu.VMEM((_CHUNK, p), f32),
               pltpu.VMEM((_CHUNK,), i32), pltpu.VMEM((_CHUNK, p), f32)]
    if rem:
        scratch += [pltpu.VMEM((rem,), i32), pltpu.VMEM((rem, p), f32)]
    scratch += [pltpu.SemaphoreType.DMA, pltpu.SemaphoreType.DMA,
                pltpu.SemaphoreType.DMA]

    @functools.partial(
        pl.kernel, mesh=mesh,
        out_type=[jax.ShapeDtypeStruct((e, p), f32),
                  jax.ShapeDtypeStruct((e, p), f32)],
        scratch_types=scratch)
    def kern(tab_hbm, src_hbm, dst_hbm, gs_hbm, gd_hbm, *scr):
        if rem:
            (i0, r0, i1, r1, ir, rr, sem0, sem1, semr) = scr
        else:
            (i0, r0, i1, r1, sem0, sem1, semr) = scr
        wid = lax.axis_index("s") * 2 + lax.axis_index("c")
        base0 = wid * per

        def table_pass(idx_hbm, out_hbm):
            def start(j, ib, rb, sem):
                b = base0 + j * _CHUNK
                pltpu.sync_copy(idx_hbm.at[pl.ds(b, _CHUNK)], ib)
                pltpu.async_copy(tab_hbm.at[ib], rb, sem)

            def finish(j, ib, rb, sem):
                pltpu.make_async_copy(tab_hbm.at[ib], rb, sem).wait()
                pltpu.sync_copy(rb, out_hbm.at[pl.ds(base0 + j * _CHUNK,
                                                     _CHUNK)])

            start(0, i0, r0, sem0)

            def body(i, carry):
                start(2 * i + 1, i1, r1, sem1)
                finish(2 * i, i0, r0, sem0)
                start(2 * i + 2, i0, r0, sem0)
                finish(2 * i + 1, i1, r1, sem1)
                return carry

            lax.fori_loop(0, nloop, body, 0)
            finish(full - 1, i0, r0, sem0)
            if rem:
                b = base0 + full * _CHUNK
                pltpu.sync_copy(idx_hbm.at[pl.ds(b, rem)], ir)
                pltpu.async_copy(tab_hbm.at[ir], rr, semr).wait()
                pltpu.sync_copy(rr, out_hbm.at[pl.ds(b, rem)])

        table_pass(src_hbm, gs_hbm)
        table_pass(dst_hbm, gd_hbm)

    return kern


@functools.lru_cache(None)
def _make_scatter(e, n, nh):
    per = e // _NW
    full = per // _CHUNK
    rem = per - full * _CHUNK
    rpt = ((n // 16) + 7) // 8 * 8
    rlast = n - 15 * rpt
    mesh = plsc.VectorSubcoreMesh(core_axis_name="c", subcore_axis_name="s")

    assert full % 2 == 1 and full >= 3
    nloop = (full - 1) // 2
    scratch = [pltpu.VMEM_SHARED((n, 128), f32),
               pltpu.VMEM((_CHUNK,), i32), pltpu.VMEM((_CHUNK, 128), f32),
               pltpu.VMEM((_CHUNK,), i32), pltpu.VMEM((_CHUNK, 128), f32)]
    if rem:
        scratch += [pltpu.VMEM((rem,), i32), pltpu.VMEM((rem, 128), f32)]
    scratch += [pltpu.SemaphoreType.DMA, pltpu.SemaphoreType.DMA,
                pltpu.SemaphoreType.DMA]

    @functools.partial(
        pl.kernel, mesh=mesh,
        out_type=jax.ShapeDtypeStruct((nh, 2, n, 128), f32),
        scratch_types=scratch)
    def kern(u_hbm, dst_hbm, z_hbm, out_hbm, *scr):
        if rem:
            (agg_sh, i0, u0, i1, u1, ir, ur, sem0, sem1, semr) = scr
        else:
            (agg_sh, i0, u0, i1, u1, sem0, sem1, semr) = scr
        c = lax.axis_index("c")
        s = lax.axis_index("s")
        wid = s * 2 + c
        base0 = wid * per

        def _rows(fn):
            @pl.when(s < 15)
            def _():
                fn(s * rpt, rpt)

            @pl.when(s == 15)
            def _():
                fn(15 * rpt, rlast)

        for h in range(nh):
            _rows(lambda r0, sz: pltpu.sync_copy(
                z_hbm.at[pl.ds(r0, sz)], agg_sh.at[pl.ds(r0, sz)]))
            plsc.subcore_barrier()

            def start(j, ib, ub, sem):
                b = base0 + j * _CHUNK
                pltpu.async_copy(dst_hbm.at[pl.ds(b, _CHUNK)], ib, sem)
                pltpu.async_copy(u_hbm.at[h, pl.ds(b, _CHUNK)], ub, sem)

            def finish(j, ib, ub, sem):
                b = base0 + j * _CHUNK
                pltpu.make_async_copy(dst_hbm.at[pl.ds(b, _CHUNK)], ib,
                                      sem).wait()
                pltpu.make_async_copy(u_hbm.at[h, pl.ds(b, _CHUNK)], ub,
                                      sem).wait()
                pltpu.sync_copy(ub, agg_sh.at[ib], add=True)

            start(0, i0, u0, sem0)

            def body(i, carry):
                start(2 * i + 1, i1, u1, sem1)
                finish(2 * i, i0, u0, sem0)
                start(2 * i + 2, i0, u0, sem0)
                finish(2 * i + 1, i1, u1, sem1)
                return carry

            lax.fori_loop(0, nloop, body, 0)
            finish(full - 1, i0, u0, sem0)
            if rem:
                b = base0 + full * _CHUNK
                pltpu.sync_copy(dst_hbm.at[pl.ds(b, rem)], ir)
                pltpu.sync_copy(u_hbm.at[h, pl.ds(b, rem)], ur)
                pltpu.sync_copy(ur, agg_sh.at[ir], add=True)
            plsc.subcore_barrier()
            _rows(lambda r0, sz: pltpu.sync_copy(
                agg_sh.at[pl.ds(r0, sz)], out_hbm.at[h, c, pl.ds(r0, sz)]))
            plsc.subcore_barrier()

    return kern


def _sc_gather(table, src, dst):
    n, p = table.shape
    e = src.shape[0]
    return _make_gather(e, n, p)(table, src, dst)


def _sc_scatter(u3, dst, zeros_pad):
    nh, e, _ = u3.shape
    n = zeros_pad.shape[0]
    return _make_scatter(e, n, nh)(u3, dst, zeros_pad)


def _pad_cols(x, width):
    return jnp.pad(x, ((0, 0), (0, width - x.shape[1])))


def _branch(v_in, e_attr, src, dst, emb_p, convs, dpu, npad, store_y):
    n = v_in.shape[0]
    e = e_attr.shape[0]
    d = emb_p["W"].shape[1]

    ss0, sq0 = _mm_stats(v_in, emb_p["W"], emb_p["b"][None, :], 2000)
    a0, c0 = _bn_affine(ss0, sq0, n, emb_p["g"][None, :],
                        emb_p["bt"][None, :])
    node = _emb_apply(v_in, emb_p["W"], emb_p["b"][None, :], a0, c0,
                      npad, 2000)

    zeros_pad = jnp.zeros((n, 128), f32)

    def _pad_rows(x):
        return jnp.pad(x, ((0, npad - x.shape[0]), (0, 0)))

    for p in convs:
        wm, ws = p["mlp"]["W"], p["screen"]["W"]
        wsrc = _pad_rows(jnp.concatenate(
            [_pad_cols(wm[:d], dpu), _pad_cols(ws[:d], dpu)], axis=1))
        wdst = _pad_rows(jnp.concatenate(
            [_pad_cols(wm[d:2 * d], dpu), _pad_cols(ws[d:2 * d], dpu)],
            axis=1))
        wc = jnp.pad(
            jnp.concatenate([_pad_cols(wm[2 * d:], dpu),
                             _pad_cols(ws[2 * d:], dpu)], axis=1),
            ((0, dpu - d), (0, 0)))
        bc = jnp.concatenate(
            [_pad_cols(p["mlp"]["b"][None, :], dpu),
             _pad_cols(p["screen"]["b"][None, :], dpu)], axis=1)
        pe = p["edgemlp"]
        we = _pad_cols(pe["W"], dpu)
        be = _pad_cols(pe["b"][None, :], dpu)
        sse, sqe = _mm_stats(e_attr, we, be, 4000)
        ae, ce0 = _bn_affine(sse, sqe, e, _pad_cols(pe["g"][None, :], dpu),
                             _pad_cols(pe["bt"][None, :], dpu))
        ce = ae * be + ce0

        gs, gd = _sc_gather(node, src, dst)
        if store_y:
            ssum, ssq, yv = _edge_stats_y(e_attr, gs, gd, wsrc, wdst, we,
                                          ae, ce, wc, bc, 4000)
        else:
            ssum, ssq = _edge_stats(e_attr, gs, gd, wsrc, wdst, we, ae, ce,
                                    wc, bc, 4000)
        mu = ssum / e
        var = ssq / e - mu * mu
        g_cat = jnp.concatenate(
            [_pad_cols(p["mlp"]["g"][None, :], dpu),
             _pad_cols(p["screen"]["g"][None, :], dpu)], axis=1)
        t_cat = jnp.concatenate(
            [_pad_cols(p["mlp"]["bt"][None, :], dpu),
             _pad_cols(p["screen"]["bt"][None, :], dpu)], axis=1)
        a_cat = g_cat / jnp.sqrt(var + 1e-5)
        c_cat = t_cat - a_cat * mu
        if store_y:
            u = _edge_update_y(yv, a_cat[:, :dpu], c_cat[:, :dpu],
                               a_cat[:, dpu:], c_cat[:, dpu:], dpu, 4000)
        else:
            u = _edge_update(e_attr, gs, gd, wsrc, wdst, we, ae, ce, wc, bc,
                             a_cat[:, :dpu], c_cat[:, :dpu],
                             a_cat[:, dpu:], c_cat[:, dpu:], dpu, 4000)
        partials = _sc_scatter(u, dst, zeros_pad)
        s6, q6 = _agg_stats(partials, 2000)
        mu6 = s6 / n
        var6 = q6 / n - mu6 * mu6
        aa = (p["bn_g"][None, :] /
              jnp.sqrt(var6[:, :d] + 1e-5))
        ca = p["bn_b"][None, :] - aa * mu6[:, :d]
        node = _node_update(partials, node, aa, ca, d, 2000)

    return node


def kernel(va, ea, edge_index_a, node_graph_a, vs, es, edge_index_s,
           node_graph_s, params):
    n_graphs = 256
    node_a = _branch(va, ea, edge_index_a[0], edge_index_a[1],
                     params["emb_a"], params["convs_a"], 128, 128, True)
    node_s = _branch(vs, es, edge_index_s[0], edge_index_s[1],
                     params["emb_s"], params["convs_s"], 192, 256, True)

    va_sum, va_cnt = _pool_sums(node_a, node_graph_a[:, None], n_graphs, 2000)
    vs_sum, vs_cnt = _pool_sums(node_s, node_graph_s[:, None], n_graphs, 2000)

    pr = params
    plist = [pr["atten"]["W"], pr["atten"]["b"][None, :],
             pr["atten"]["g"][None, :], pr["atten"]["bt"][None, :],
             pr["atten_bn_g"][None, :], pr["atten_bn_b"][None, :]]
    for fc in pr["fcs"]:
        plist += [fc["W"], fc["b"][None, :], fc["g"][None, :],
                  fc["bt"][None, :]]
    for nm in ("pred_adsb", "pred_site"):
        plist += [pr[nm]["W"], pr[nm]["b"][None, :], pr[nm]["g"][None, :],
                  pr[nm]["bt"][None, :]]
    plist += [pr["pred_target"]["W"], pr["pred_target"]["b"][None, :]]

    return _head(va_sum, va_cnt, vs_sum, vs_cnt, 110, 150, plist)
